# baseline (speedup 1.0000x reference)
import functools

import jax
import jax.numpy as jnp
from jax import lax
from jax.experimental import pallas as pl
from jax.experimental.pallas import tpu as pltpu
from jax.experimental.pallas import tpu_sc as plsc

NUM_CLASSES = 1000
N_CTX = 16
CTX_DIM = 512
SEQ_LEN = 77
SUFFIX_LEN = SEQ_LEN - 1 - N_CTX
BATCH = 4096

_INFO = plsc.get_sparse_core_info()
_NC = _INFO.num_cores
_NW = _INFO.num_cores * _INFO.num_subcores
_B_PER_W = BATCH // _NW
_NSLOT = 3


@functools.partial(
    pl.kernel,
    out_type=jax.ShapeDtypeStruct((BATCH, SEQ_LEN, CTX_DIM), jnp.float32),
    mesh=plsc.VectorSubcoreMesh(core_axis_name="c", subcore_axis_name="s"),
    scratch_types=[
        pltpu.VMEM((_B_PER_W,), jnp.int32),
        [pltpu.VMEM((SEQ_LEN, CTX_DIM), jnp.float32) for _ in range(_NSLOT)],
        [pltpu.SemaphoreType.DMA for _ in range(_NSLOT)],
        [pltpu.SemaphoreType.DMA for _ in range(_NSLOT)],
    ],
)
def _gather(label_hbm, comb_hbm, out_hbm, idx_v, rowbufs, gsems, ssems):
    wid = lax.axis_index("s") * _NC + lax.axis_index("c")
    base = wid * _B_PER_W
    pltpu.sync_copy(label_hbm.at[pl.ds(base, _B_PER_W)], idx_v)

    store_pending = [None] * _NSLOT
    gather_pending = [None] * _NSLOT
    lbls = None

    def fire_gather(i):
        nonlocal lbls
        s = i % _NSLOT
        if store_pending[s] is not None:
            store_pending[s].wait()
            store_pending[s] = None
        if i % 16 == 0:
            lbls = idx_v[pl.ds(i, 16)]
        lbl = lbls[i % 16]
        gather_pending[s] = pltpu.async_copy(comb_hbm.at[lbl], rowbufs[s],
                                             gsems[s])

    fire_gather(0)
    for i in range(_B_PER_W):
        s = i % _NSLOT
        if i + 1 < _B_PER_W:
            fire_gather(i + 1)
        gather_pending[s].wait()
        store_pending[s] = pltpu.async_copy(
            rowbufs[s], out_hbm.at[base + i], ssems[s])
    for c in store_pending:
        if c is not None:
            c.wait()


_CB = 40


def _concat_body(pref_ref, ctx_ref, suf_ref, out_ref):
    out_ref[...] = jnp.concatenate(
        [jnp.broadcast_to(pref_ref[...], (_CB, 1, CTX_DIM)), ctx_ref[...],
         suf_ref[...]], axis=1)


_concat = pl.pallas_call(
    _concat_body,
    grid=(NUM_CLASSES // _CB,),
    in_specs=[
        pl.BlockSpec((1, 1, CTX_DIM), lambda i: (0, 0, 0)),
        pl.BlockSpec((_CB, N_CTX, CTX_DIM), lambda i: (i, 0, 0)),
        pl.BlockSpec((_CB, SUFFIX_LEN, CTX_DIM), lambda i: (i, 0, 0)),
    ],
    out_specs=pl.BlockSpec((_CB, SEQ_LEN, CTX_DIM), lambda i: (i, 0, 0)),
    out_shape=jax.ShapeDtypeStruct((NUM_CLASSES, SEQ_LEN, CTX_DIM),
                                   jnp.float32),
)


def kernel(label, cls_ctx, token_prefix, token_suffix):
    comb = _concat(token_prefix, cls_ctx, token_suffix)
    return _gather(label.astype(jnp.int32), comb)


# speedup vs baseline: 1.0029x; 1.0029x over previous
import functools

import jax
import jax.numpy as jnp
from jax import lax
from jax.experimental import pallas as pl
from jax.experimental.pallas import tpu as pltpu
from jax.experimental.pallas import tpu_sc as plsc

NUM_CLASSES = 1000
N_CTX = 16
CTX_DIM = 512
SEQ_LEN = 77
SUFFIX_LEN = SEQ_LEN - 1 - N_CTX
BATCH = 4096

_INFO = plsc.get_sparse_core_info()
_NC = _INFO.num_cores
_NW = _INFO.num_cores * _INFO.num_subcores
_B_PER_W = BATCH // _NW
_NSLOT = 3


@functools.partial(
    pl.kernel,
    out_type=jax.ShapeDtypeStruct((BATCH, SEQ_LEN, CTX_DIM), jnp.float32),
    mesh=plsc.VectorSubcoreMesh(core_axis_name="c", subcore_axis_name="s"),
    scratch_types=[
        pltpu.VMEM((_B_PER_W,), jnp.int32),
        [pltpu.VMEM((SEQ_LEN, CTX_DIM), jnp.float32) for _ in range(_NSLOT)],
        [pltpu.SemaphoreType.DMA for _ in range(_NSLOT)],
        [pltpu.SemaphoreType.DMA for _ in range(_NSLOT)],
    ],
)
def _gather(label_hbm, comb_hbm, out_hbm, idx_v, rowbufs, gsems, ssems):
    wid = lax.axis_index("s") * _NC + lax.axis_index("c")
    base = wid * _B_PER_W
    pltpu.sync_copy(label_hbm.at[pl.ds(base, _B_PER_W)], idx_v)

    store_pending = [None] * _NSLOT
    gather_pending = [None] * _NSLOT
    lbls = None

    def fire_gather(i):
        nonlocal lbls
        s = i % _NSLOT
        if store_pending[s] is not None:
            store_pending[s].wait()
            store_pending[s] = None
        if i % 16 == 0:
            lbls = idx_v[pl.ds(i, 16)]
        lbl = lbls[i % 16]
        gather_pending[s] = pltpu.async_copy(comb_hbm.at[lbl], rowbufs[s],
                                             gsems[s])

    fire_gather(0)
    for i in range(_B_PER_W):
        s = i % _NSLOT
        if i + 1 < _B_PER_W:
            fire_gather(i + 1)
        gather_pending[s].wait()
        store_pending[s] = pltpu.async_copy(
            rowbufs[s], out_hbm.at[base + i], ssems[s])
    for c in store_pending:
        if c is not None:
            c.wait()


_CB = 40


def _concat_body(pref_ref, ctx_ref, suf_ref, out_ref):
    out_ref[:, 0:1, :] = jnp.broadcast_to(pref_ref[...], (_CB, 1, CTX_DIM))
    out_ref[:, 1:1 + N_CTX, :] = ctx_ref[...]
    out_ref[:, 1 + N_CTX:, :] = suf_ref[...]


_concat = pl.pallas_call(
    _concat_body,
    grid=(NUM_CLASSES // _CB,),
    in_specs=[
        pl.BlockSpec((1, 1, CTX_DIM), lambda i: (0, 0, 0)),
        pl.BlockSpec((_CB, N_CTX, CTX_DIM), lambda i: (i, 0, 0)),
        pl.BlockSpec((_CB, SUFFIX_LEN, CTX_DIM), lambda i: (i, 0, 0)),
    ],
    out_specs=pl.BlockSpec((_CB, SEQ_LEN, CTX_DIM), lambda i: (i, 0, 0)),
    out_shape=jax.ShapeDtypeStruct((NUM_CLASSES, SEQ_LEN, CTX_DIM),
                                   jnp.float32),
)


def kernel(label, cls_ctx, token_prefix, token_suffix):
    comb = _concat(token_prefix, cls_ctx, token_suffix)
    return _gather(label.astype(jnp.int32), comb)
